# Initial kernel scaffold; baseline (speedup 1.0000x reference)
import jax
import jax.numpy as jnp
from jax.experimental import pallas as pl


def kernel(x, edge_index, dist_attn, path_attn, ln_gamma, ln_beta, Wqkv, bqkv, Wout, bout):
    raise NotImplementedError("write your pallas kernel here")


# speedup vs baseline: 2.6152x; 2.6152x over previous
import functools

import jax
import jax.numpy as jnp
from jax import lax
from jax.experimental import pallas as pl
from jax.experimental.pallas import tpu as pltpu
from jax.experimental.pallas import tpu_sc as plsc

N = 10000
E = 320000
D = 128
H = 8
DH = D // H
L = 16

NC = 2
NS = 16
NW = NC * NS

N_PAD = 10240
E_PAD = 327680
CH = 128
EPT = E_PAD // NW
NCHUNK = EPT // CH
RPS = N_PAD // NS
NFL = RPS // CH

_mesh = plsc.VectorSubcoreMesh(core_axis_name="c", subcore_axis_name="s")


def _ln_qkv_body(x_ref, g_ref, b_ref, w_ref, bias_ref, q_ref, k_ref, v_ref):
    x = x_ref[...]
    mu = jnp.mean(x, axis=-1, keepdims=True)
    var = jnp.mean((x - mu) ** 2, axis=-1, keepdims=True)
    h = (x - mu) / jnp.sqrt(var + 1e-5) * g_ref[...] + b_ref[...]
    qkv = lax.dot_general(h, w_ref[...], (((1,), (1,)), ((), ())),
                          preferred_element_type=jnp.float32) + bias_ref[...]
    q_ref[...] = qkv[:, :D] * (D ** -0.5)
    k_ref[...] = qkv[:, D:2 * D]
    v_ref[...] = qkv[:, 2 * D:]


def _ln_qkv(x_pad, ln_gamma, ln_beta, Wqkv, bqkv):
    RB = 2560
    grid = (N_PAD // RB,)
    return pl.pallas_call(
        _ln_qkv_body,
        grid=grid,
        in_specs=[
            pl.BlockSpec((RB, D), lambda i: (i, 0)),
            pl.BlockSpec((1, D), lambda i: (0, 0)),
            pl.BlockSpec((1, D), lambda i: (0, 0)),
            pl.BlockSpec((3 * D, D), lambda i: (0, 0)),
            pl.BlockSpec((1, 3 * D), lambda i: (0, 0)),
        ],
        out_specs=[
            pl.BlockSpec((RB, D), lambda i: (i, 0)),
            pl.BlockSpec((RB, D), lambda i: (i, 0)),
            pl.BlockSpec((RB, D), lambda i: (i, 0)),
        ],
        out_shape=[jax.ShapeDtypeStruct((N_PAD, D), jnp.float32)] * 3,
    )(x_pad, ln_gamma.reshape(1, D), ln_beta.reshape(1, D), Wqkv,
      bqkv.reshape(1, 3 * D))


@functools.partial(
    pl.kernel,
    out_type=[
        jax.ShapeDtypeStruct((NW, NCHUNK, H, CH), jnp.float32),
        jax.ShapeDtypeStruct((NC * N_PAD * H,), jnp.float32),
    ],
    mesh=_mesh,
    compiler_params=pltpu.CompilerParams(needs_layout_passes=False),
    scratch_types=[
        pltpu.VMEM((CH,), jnp.int32),
        pltpu.VMEM((CH,), jnp.int32),
        pltpu.VMEM((H, CH), jnp.int32),
        pltpu.VMEM((CH, D), jnp.float32),
        pltpu.VMEM((CH, D), jnp.float32),
        pltpu.VMEM((H, CH), jnp.float32),
        pltpu.VMEM((H, CH), jnp.float32),
        pltpu.VMEM((H, CH), jnp.float32),
        pltpu.VMEM((RPS * H,), jnp.float32),
        pltpu.VMEM_SHARED((N_PAD * H,), jnp.float32),
        pltpu.SemaphoreType.DMA,
        pltpu.SemaphoreType.DMA,
    ],
)
def _edge_logits(q_hbm, k_hbm, src_hbm, dst_hbm, dist_hbm, path_hbm,
                 aexp_hbm, sparts_hbm,
                 src_v, dst_v, idx8_v, qrows, krows, dist_v, path_v, aexp_h,
                 stage_v, s_sh, sem0, sem1):
    cid = lax.axis_index("c")
    sid = lax.axis_index("s")
    wid = sid * NC + cid

    def _zfill(i, carry):
        stage_v[pl.ds(i * L, L)] = jnp.zeros((L,), jnp.float32)
        return carry
    lax.fori_loop(0, RPS * H // L, _zfill, 0)
    pltpu.sync_copy(stage_v, s_sh.at[pl.ds(sid * (RPS * H), RPS * H)])
    plsc.subcore_barrier()

    def _chunk(g, carry):
        base = wid * EPT + g * CH
        pltpu.sync_copy(src_hbm.at[pl.ds(base, CH)], src_v)
        pltpu.sync_copy(dst_hbm.at[pl.ds(base, CH)], dst_v)
        pltpu.sync_copy(dist_hbm.at[wid, g], dist_v)
        pltpu.sync_copy(path_hbm.at[wid, g], path_v)
        cp_q = pltpu.async_copy(q_hbm.at[src_v], qrows, sem0)
        cp_k = pltpu.async_copy(k_hbm.at[dst_v], krows, sem1)

        def _mkidx(j, inner):
            d16 = dst_v[pl.ds(j * L, L)]
            for h in range(H):
                idx8_v[h, pl.ds(j * L, L)] = d16 * H + h
            return inner
        lax.fori_loop(0, CH // L, _mkidx, 0)
        cp_q.wait()
        cp_k.wait()

        def _e16(t, inner):
            lanes = lax.iota(jnp.int32, L)
            rows = t * L + lanes
            for h in range(H):
                hrow = jnp.full((L,), h, jnp.int32)
                acc = (plsc.load_gather(dist_v, [hrow, rows]) +
                       plsc.load_gather(path_v, [hrow, rows]))
                for dd in range(DH):
                    col = jnp.full((L,), h * DH + dd, jnp.int32)
                    acc = acc + (plsc.load_gather(qrows, [rows, col]) *
                                 plsc.load_gather(krows, [rows, col]))
                plsc.store_scatter(aexp_h, [hrow, rows], jnp.exp(acc))
            return inner
        lax.fori_loop(0, CH // L, _e16, 0)

        pltpu.sync_copy(aexp_h, aexp_hbm.at[wid, g])
        for h in range(H):
            pltpu.sync_copy(aexp_h.at[h], s_sh.at[idx8_v.at[h]], add=True)
        return carry
    lax.fori_loop(0, NCHUNK, _chunk, 0)

    plsc.subcore_barrier()
    pltpu.sync_copy(s_sh.at[pl.ds(sid * (RPS * H), RPS * H)], stage_v)
    pltpu.sync_copy(stage_v,
                    sparts_hbm.at[pl.ds((cid * NS + sid) * (RPS * H),
                                        RPS * H)])


@functools.partial(
    pl.kernel,
    out_type=jax.ShapeDtypeStruct((NC, NS, RPS, D), jnp.float32),
    mesh=_mesh,
    compiler_params=pltpu.CompilerParams(needs_layout_passes=False),
    scratch_types=[
        pltpu.VMEM((CH,), jnp.int32),
        pltpu.VMEM((1, CH), jnp.int32),
        pltpu.VMEM((1, CH), jnp.int32),
        pltpu.VMEM((CH, D), jnp.float32),
        pltpu.VMEM((H, CH), jnp.float32),
        pltpu.VMEM_SHARED((N_PAD, D), jnp.float32),
        pltpu.SemaphoreType.DMA,
    ],
)
def _edge_agg(v_hbm, src_hbm, dst_hbm, aexp_hbm,
              agg_hbm,
              src_v, dst2_v, ridx_v, vrows, aexp_h, agg_sh, sem0):
    cid = lax.axis_index("c")
    sid = lax.axis_index("s")
    wid = sid * NC + cid

    def _zfill(e, carry):
        for j in range(D // L):
            vrows[e, pl.ds(j * L, L)] = jnp.zeros((L,), jnp.float32)
        return carry
    lax.fori_loop(0, CH, _zfill, 0)
    for c in range(NFL):
        def _ridx(j, carry, _c=c):
            ridx_v[0, pl.ds(j * L, L)] = (
                sid * RPS + _c * CH + j * L + lax.iota(jnp.int32, L))
            return carry
        lax.fori_loop(0, CH // L, _ridx, 0)
        pltpu.sync_copy(vrows, agg_sh.at[ridx_v.at[0]])
    plsc.subcore_barrier()

    def _chunk(g, carry):
        base = wid * EPT + g * CH
        pltpu.sync_copy(src_hbm.at[pl.ds(base, CH)], src_v)
        pltpu.sync_copy(dst_hbm.at[pl.ds(base, CH)], dst2_v.at[0])
        pltpu.sync_copy(aexp_hbm.at[wid, g], aexp_h)
        cp_v = pltpu.async_copy(v_hbm.at[src_v], vrows, sem0)
        cp_v.wait()

        def _scale(t, inner):
            for ll in range(L):
                e = t * L + ll
                for h in range(H):
                    w = plsc.load_gather(
                        aexp_h,
                        [jnp.full((L,), h, jnp.int32),
                         jnp.zeros((L,), jnp.int32) + e])
                    vrows[e, pl.ds(h * DH, DH)] = (
                        vrows[e, pl.ds(h * DH, DH)] * w)
            return inner
        lax.fori_loop(0, CH // L, _scale, 0)

        pltpu.sync_copy(vrows, agg_sh.at[dst2_v.at[0]], add=True)
        return carry
    lax.fori_loop(0, NCHUNK, _chunk, 0)

    plsc.subcore_barrier()
    for c in range(NFL):
        def _ridx(j, carry, _c=c):
            ridx_v[0, pl.ds(j * L, L)] = (
                sid * RPS + _c * CH + j * L + lax.iota(jnp.int32, L))
            return carry
        lax.fori_loop(0, CH // L, _ridx, 0)
        pltpu.sync_copy(agg_sh.at[ridx_v.at[0]], vrows)
        pltpu.sync_copy(vrows, agg_hbm.at[cid, sid, pl.ds(c * CH, CH), :])


def _out_proj_body(a0_ref, a1_ref, s0_ref, s1_ref, x_ref, ex_ref, w_ref,
                   b_ref, o_ref):
    r = 1.0 / jnp.maximum(s0_ref[...] + s1_ref[...], 1e-30)
    er = lax.dot_general(r, ex_ref[...], (((1,), (0,)), ((), ())),
                         preferred_element_type=jnp.float32)
    agg = (a0_ref[...] + a1_ref[...]) * er
    o_ref[...] = (x_ref[...] +
                  lax.dot_general(agg, w_ref[...], (((1,), (1,)), ((), ())),
                                  preferred_element_type=jnp.float32) +
                  b_ref[...])


def _out_proj(agg0, agg1, s0, s1, x, expand, Wout, bout):
    RB = 2000
    grid = (N // RB,)
    return pl.pallas_call(
        _out_proj_body,
        grid=grid,
        in_specs=[
            pl.BlockSpec((RB, D), lambda i: (i, 0)),
            pl.BlockSpec((RB, D), lambda i: (i, 0)),
            pl.BlockSpec((RB, H), lambda i: (i, 0)),
            pl.BlockSpec((RB, H), lambda i: (i, 0)),
            pl.BlockSpec((RB, D), lambda i: (i, 0)),
            pl.BlockSpec((H, D), lambda i: (0, 0)),
            pl.BlockSpec((D, D), lambda i: (0, 0)),
            pl.BlockSpec((1, D), lambda i: (0, 0)),
        ],
        out_specs=pl.BlockSpec((RB, D), lambda i: (i, 0)),
        out_shape=jax.ShapeDtypeStruct((N, D), jnp.float32),
    )(agg0, agg1, s0, s1, x, expand, Wout, bout.reshape(1, D))


def kernel(x, edge_index, dist_attn, path_attn, ln_gamma, ln_beta,
           Wqkv, bqkv, Wout, bout):
    x_pad = jnp.pad(x, ((0, N_PAD - N), (0, 0)))
    src = jnp.concatenate(
        [edge_index[0], jnp.full((E_PAD - E,), N, jnp.int32)])
    dst = jnp.concatenate(
        [edge_index[1], jnp.full((E_PAD - E,), N, jnp.int32)])
    dist_r = jnp.pad(dist_attn, ((0, E_PAD - E), (0, 0))).reshape(
        NW, NCHUNK, CH, H).transpose(0, 1, 3, 2)
    path_r = jnp.pad(path_attn, ((0, E_PAD - E), (0, 0))).reshape(
        NW, NCHUNK, CH, H).transpose(0, 1, 3, 2)
    expand = jnp.repeat(jnp.eye(H, dtype=jnp.float32), DH, axis=1)

    q, k, v = _ln_qkv(x_pad, ln_gamma, ln_beta, Wqkv, bqkv)
    aexp, sparts = _edge_logits(q, k, src, dst, dist_r, path_r)
    agg = _edge_agg(v, src, dst, aexp)
    aggr = agg.reshape(NC, N_PAD, D)
    s0 = sparts[:N_PAD * H].reshape(N_PAD, H)[:N]
    s1 = sparts[N_PAD * H:].reshape(N_PAD, H)[:N]
    return _out_proj(aggr[0, :N], aggr[1, :N], s0, s1, x, expand,
                     Wout, bout)
